# baseline (speedup 1.0000x reference)
import functools

import jax
import jax.numpy as jnp
from jax import lax
from jax.experimental import pallas as pl
from jax.experimental.pallas import tpu as pltpu
from jax.experimental.pallas import tpu_sc as plsc

_VOCAB = 1000
_EMB = 32
_T = 8
_B = 4096
_NROWS = _B * _T

_VPAD = 1024
_NW = 32
_ROWS_PER_W = _NROWS // _NW
_CHUNK = 32
_NCHUNK = _ROWS_PER_W // _CHUNK


def _table_kernel(emb_ref, posw_ref, w_ref, out_ref):
    t = pl.program_id(0)
    ew = jnp.dot(emb_ref[:], w_ref[:], preferred_element_type=jnp.float32)
    out_ref[:] = ew + posw_ref[t, :][None, :]


def _build_table8(embedding, positional_embedding, lm_head_w, lm_head_b):
    posw = positional_embedding @ lm_head_w + lm_head_b[None, :]
    return pl.pallas_call(
        _table_kernel,
        grid=(_T,),
        in_specs=[
            pl.BlockSpec((_VOCAB, _EMB), lambda t: (0, 0)),
            pl.BlockSpec((_T, _VOCAB), lambda t: (0, 0)),
            pl.BlockSpec((_EMB, _VOCAB), lambda t: (0, 0)),
        ],
        out_specs=pl.BlockSpec((_VOCAB, _VOCAB), lambda t: (t, 0)),
        out_shape=jax.ShapeDtypeStruct((_T * _VOCAB, _VOCAB), jnp.float32),
    )(embedding, posw, lm_head_w)


def _sc_gather_body(tab_ref, jidx_ref, out_ref, idx_v, buf0, buf1,
                    gsem0, gsem1, osem0, osem1):
    wid = lax.axis_index("s") * 2 + lax.axis_index("c")
    base = wid * _ROWS_PER_W

    pltpu.sync_copy(jidx_ref.at[wid], idx_v)

    bufs = (buf0, buf1)
    gsems = (gsem0, gsem1)
    osems = (osem0, osem1)

    gathers = [None] * _NCHUNK
    outs = [None] * _NCHUNK

    gathers[0] = pltpu.async_copy(tab_ref.at[idx_v.at[0]], bufs[0], gsems[0])
    for c in range(_NCHUNK):
        b = c & 1
        gathers[c].wait()
        if c + 1 < _NCHUNK:
            if c >= 1:
                outs[c - 1].wait()
            gathers[c + 1] = pltpu.async_copy(
                tab_ref.at[idx_v.at[c + 1]], bufs[1 - b], gsems[1 - b]
            )
        outs[c] = pltpu.async_copy(
            bufs[b],
            out_ref.at[pl.ds(base + c * _CHUNK, _CHUNK)],
            osems[b],
        )
    outs[_NCHUNK - 2].wait()
    outs[_NCHUNK - 1].wait()


def _sc_gather(table8, jidx):
    mesh = plsc.VectorSubcoreMesh(core_axis_name="c", subcore_axis_name="s")
    fn = functools.partial(
        pl.kernel,
        out_type=jax.ShapeDtypeStruct((_NROWS, _VOCAB), jnp.float32),
        mesh=mesh,
        scratch_types=[
            pltpu.VMEM((_NCHUNK, _CHUNK), jnp.int32),
            pltpu.VMEM((_CHUNK, _VOCAB), jnp.float32),
            pltpu.VMEM((_CHUNK, _VOCAB), jnp.float32),
            pltpu.SemaphoreType.DMA,
            pltpu.SemaphoreType.DMA,
            pltpu.SemaphoreType.DMA,
            pltpu.SemaphoreType.DMA,
        ],
        compiler_params=pltpu.CompilerParams(use_tc_tiling_on_sc=False),
    )(_sc_gather_body)
    return fn(table8, jidx)


@jax.jit
def kernel(idx, embedding, positional_embedding, lm_head_w, lm_head_b):
    B, T = idx.shape
    table8 = _build_table8(embedding, positional_embedding, lm_head_w,
                           lm_head_b)
    flat = idx.reshape(_NROWS).astype(jnp.int32)
    j = flat + _VOCAB * (jax.lax.iota(jnp.int32, _NROWS) % _T)
    jidx = j.reshape(_NW, _NCHUNK, _CHUNK)
    out = _sc_gather(table8, jidx)
    return out.reshape(B, T, _VOCAB)
